# baseline (speedup 1.0000x reference)
import functools

import jax
import jax.numpy as jnp
from jax import lax
from jax.experimental import pallas as pl
from jax.experimental.pallas import tpu as pltpu
from jax.experimental.pallas import tpu_sc as plsc

_NC, _NS = 2, 16
_NW = _NC * _NS
_CH = 64
_LANES = 16


def _sc_body(S, D, B, x_hbm, pe_hbm, out_hbm, xbuf, pebuf):
    s_per_w = S // _NW
    n_ch = s_per_w // _CH
    w = lax.axis_index("s") * _NC + lax.axis_index("c")
    s0 = w * s_per_w
    for ic in range(n_ch):
        pe_off = (s0 + ic * _CH) * D
        pltpu.sync_copy(pe_hbm.at[pl.ds(pe_off, _CH * D)], pebuf)
        for b in range(B):
            x_off = (b * S + s0 + ic * _CH) * D
            pltpu.sync_copy(x_hbm.at[pl.ds(x_off, _CH * D)], xbuf)

            @plsc.parallel_loop(0, _CH * D, _LANES, unroll=8)
            def _(i):
                plsc.addupdate(xbuf.at[pl.ds(i, _LANES)], pebuf[pl.ds(i, _LANES)])

            pltpu.sync_copy(xbuf, out_hbm.at[pl.ds(x_off, _CH * D)])


def kernel(x, pe):
    B, S, D = x.shape
    mesh = plsc.VectorSubcoreMesh(core_axis_name="c", subcore_axis_name="s")
    run = pl.kernel(
        functools.partial(_sc_body, S, D, B),
        out_type=jax.ShapeDtypeStruct((B * S * D,), x.dtype),
        mesh=mesh,
        scratch_types=[
            pltpu.VMEM((_CH * D,), jnp.float32),
            pltpu.VMEM((_CH * D,), jnp.float32),
        ],
    )
    out = run(x.reshape(-1), pe[:S].reshape(-1))
    return out.reshape(B, S, D)


# speedup vs baseline: 1.1906x; 1.1906x over previous
import functools

import jax
import jax.numpy as jnp
from jax import lax
from jax.experimental import pallas as pl
from jax.experimental.pallas import tpu as pltpu
from jax.experimental.pallas import tpu_sc as plsc

_NC, _NS = 2, 16
_NW = _NC * _NS
_CH = 32
_LANES = 16
_NBUF = 3


def _sc_body(S, D, B, x_hbm, pe_hbm, out_hbm,
             xb0, xb1, xb2, pb0, pb1, lsems, ssems, psems):
    xbufs = (xb0, xb1, xb2)
    pbufs = (pb0, pb1)
    s_per_w = S // _NW
    n_ch = s_per_w // _CH
    n_steps = n_ch * B
    w = lax.axis_index("s") * _NC + lax.axis_index("c")
    s0 = w * s_per_w

    def x_off(t):
        ic, b = divmod(t, B)
        return (b * S + s0 + ic * _CH) * D

    load_d = [None] * _NBUF
    store_d = [None] * _NBUF

    def start_load(t):
        buf = t % _NBUF
        if store_d[buf] is not None:
            store_d[buf].wait()
            store_d[buf] = None
        load_d[buf] = pltpu.async_copy(
            x_hbm.at[pl.ds(x_off(t), _CH * D)], xbufs[buf], lsems.at[buf])

    def start_pe(ic):
        return pltpu.async_copy(
            pe_hbm.at[pl.ds((s0 + ic * _CH) * D, _CH * D)],
            pbufs[ic % 2], psems.at[ic % 2])

    pe_d = [start_pe(0), None]
    start_load(0)
    start_load(1)

    for t in range(n_steps):
        ic, b = divmod(t, B)
        buf = t % _NBUF
        if b == 0:
            pe_d[ic % 2].wait()
            if ic + 1 < n_ch:
                pe_d[(ic + 1) % 2] = start_pe(ic + 1)
        load_d[buf].wait()
        xbuf = xbufs[buf]
        pbuf = pbufs[ic % 2]

        @plsc.parallel_loop(0, _CH * D, _LANES, unroll=8)
        def _(i):
            plsc.addupdate(xbuf.at[pl.ds(i, _LANES)], pbuf[pl.ds(i, _LANES)])

        store_d[buf] = pltpu.async_copy(
            xbuf, out_hbm.at[pl.ds(x_off(t), _CH * D)], ssems.at[buf])
        if t + 2 < n_steps:
            start_load(t + 2)

    for d in store_d:
        if d is not None:
            d.wait()


def kernel(x, pe):
    B, S, D = x.shape
    mesh = plsc.VectorSubcoreMesh(core_axis_name="c", subcore_axis_name="s")
    run = pl.kernel(
        functools.partial(_sc_body, S, D, B),
        out_type=jax.ShapeDtypeStruct((B * S * D,), x.dtype),
        mesh=mesh,
        scratch_types=[
            pltpu.VMEM((_CH * D,), jnp.float32),
            pltpu.VMEM((_CH * D,), jnp.float32),
            pltpu.VMEM((_CH * D,), jnp.float32),
            pltpu.VMEM((_CH * D,), jnp.float32),
            pltpu.VMEM((_CH * D,), jnp.float32),
            pltpu.SemaphoreType.DMA((_NBUF,)),
            pltpu.SemaphoreType.DMA((_NBUF,)),
            pltpu.SemaphoreType.DMA((2,)),
        ],
    )
    out = run(x.reshape(-1), pe[:S].reshape(-1))
    return out.reshape(B, S, D)


# speedup vs baseline: 3.5614x; 2.9913x over previous
import functools

import jax
import jax.numpy as jnp
from jax import lax
from jax.experimental import pallas as pl
from jax.experimental.pallas import tpu as pltpu
from jax.experimental.pallas import tpu_sc as plsc

_NC, _NS = 2, 16
_NW = _NC * _NS
_CH = 32
_LANES = 16
_NBUF = 3


def _sc_body(S, D, B, x_hbm, pe_hbm, out_hbm,
             xb0, xb1, xb2, pb0, pb1, lsems, ssems, psems):
    xbufs = (xb0, xb1, xb2)
    pbufs = (pb0, pb1)
    s_per_w = S // _NW
    n_ch = s_per_w // _CH
    n_steps = n_ch * B
    w = lax.axis_index("s") * _NC + lax.axis_index("c")
    s0 = w * s_per_w

    load_d = [None] * _NBUF
    store_d = [None] * _NBUF

    def start_load(t):
        ic, b = divmod(t, B)
        buf = t % _NBUF
        if store_d[buf] is not None:
            store_d[buf].wait()
            store_d[buf] = None
        load_d[buf] = pltpu.async_copy(
            x_hbm.at[b, pl.ds(s0 + ic * _CH, _CH)], xbufs[buf],
            lsems.at[buf])

    def start_pe(ic):
        return pltpu.async_copy(
            pe_hbm.at[pl.ds(s0 + ic * _CH, _CH)],
            pbufs[ic % 2], psems.at[ic % 2])

    pe_d = [start_pe(0), None]
    start_load(0)
    start_load(1)

    for t in range(n_steps):
        ic, b = divmod(t, B)
        buf = t % _NBUF
        if b == 0:
            pe_d[ic % 2].wait()
            if ic + 1 < n_ch:
                pe_d[(ic + 1) % 2] = start_pe(ic + 1)
        load_d[buf].wait()
        xbuf = xbufs[buf]
        pbuf = pbufs[ic % 2]

        @plsc.parallel_loop(0, _CH, 1)
        def _(i):
            @plsc.parallel_loop(0, D, _LANES, unroll=8)
            def _(k):
                plsc.addupdate(xbuf.at[i, pl.ds(k, _LANES)],
                               pbuf[i, pl.ds(k, _LANES)])

        store_d[buf] = pltpu.async_copy(
            xbuf, out_hbm.at[b, pl.ds(s0 + ic * _CH, _CH)], ssems.at[buf])
        if t + 2 < n_steps:
            start_load(t + 2)

    for d in store_d:
        if d is not None:
            d.wait()


def kernel(x, pe):
    B, S, D = x.shape
    mesh = plsc.VectorSubcoreMesh(core_axis_name="c", subcore_axis_name="s")
    run = pl.kernel(
        functools.partial(_sc_body, S, D, B),
        out_type=jax.ShapeDtypeStruct((B, S, D), x.dtype),
        mesh=mesh,
        scratch_types=[
            pltpu.VMEM((_CH, D), jnp.float32),
            pltpu.VMEM((_CH, D), jnp.float32),
            pltpu.VMEM((_CH, D), jnp.float32),
            pltpu.VMEM((_CH, D), jnp.float32),
            pltpu.VMEM((_CH, D), jnp.float32),
            pltpu.SemaphoreType.DMA((_NBUF,)),
            pltpu.SemaphoreType.DMA((_NBUF,)),
            pltpu.SemaphoreType.DMA((2,)),
        ],
    )
    return run(x, pe)
